# baseline (speedup 1.0000x reference)
import jax
import jax.numpy as jnp
from jax import lax
from jax.experimental import pallas as pl
from jax.experimental.pallas import tpu as pltpu
from jax.experimental.pallas import tpu_sc as plsc

NEG_SLOPE = 0.1
_INTERP = False

NC = 2
NS = 16
K = 64


def _t1_body(x_ref, wft_ref, b_ref, alrl_ref, alrr_ref, feat_ref, el_ref, er_ref):
    xb = x_ref[...]
    feat = jnp.dot(xb, wft_ref[...], preferred_element_type=jnp.float32) + b_ref[...]
    feat_ref[...] = feat
    el_ref[...] = jnp.dot(feat, alrl_ref[...], preferred_element_type=jnp.float32)
    er_ref[...] = jnp.dot(feat, alrr_ref[...], preferred_element_type=jnp.float32)


def _t2_body(rst_ref, s_ref, sel_ref, bd_ref, bout_ref, out_ref):
    rst = rst_ref[0] + rst_ref[1]
    sv = s_ref[0] + s_ref[1]
    dexp = jnp.dot(sv, sel_ref[...], preferred_element_type=jnp.float32)
    rstn = rst / (dexp + 1e-9)
    out_ref[...] = (
        jnp.dot(rstn, bd_ref[...], preferred_element_type=jnp.float32) + bout_ref[...]
    )


def _sc_body(sdw_hbm, el_hbm, er_hbm, feat_hbm,
             rst_out, s_out,
             bufs0, bufs1, zA, zB,
             rst_acc, s_acc, sems0, sems1):
    N = el_hbm.shape[0]
    D = feat_hbm.shape[1]
    CH = sdw_hbm.shape[2]
    stripe = (N // NS) // 8 * 8
    rem = N - stripe * NS
    zrows = zA.shape[0]
    nz = stripe // zrows
    c = lax.axis_index("c")
    s = lax.axis_index("s")
    base = s * stripe

    def zrow(i, _):
        for j in range(D // 16):
            zA[i, pl.ds(16 * j, 16)] = jnp.zeros((16,), jnp.float32)
        zB[i, :] = jnp.zeros((16,), jnp.float32)
        return 0

    lax.fori_loop(0, zrows, zrow, 0)
    for r in range(nz):
        pltpu.sync_copy(zA, rst_acc.at[pl.ds(base + r * zrows, zrows)])
        pltpu.sync_copy(zB, s_acc.at[pl.ds(base + r * zrows, zrows)])

    @pl.when(s == NS - 1)
    def _zero_tail():
        pltpu.sync_copy(zA.at[pl.ds(0, rem)], rst_acc.at[pl.ds(NS * stripe, rem)])
        pltpu.sync_copy(zB.at[pl.ds(0, rem)], s_acc.at[pl.ds(NS * stripe, rem)])

    plsc.subcore_barrier()

    def issue(ci, bufs, sems):
        sdw_v, A_v, B_v, EE_v, F_v = bufs
        semA, semB, semF, sem_s, sem_m = sems

        @pl.when(ci >= 2)
        def _drain():
            pltpu.make_async_copy(EE_v, s_acc.at[sdw_v.at[1]], sem_s).wait()
            pltpu.make_async_copy(F_v, rst_acc.at[sdw_v.at[1]], sem_m).wait()

        @pl.when(ci < CH)
        def _fetch():
            pltpu.sync_copy(sdw_hbm.at[c, s, ci], sdw_v)
            pltpu.async_copy(el_hbm.at[sdw_v.at[0]], A_v, semA)
            pltpu.async_copy(er_hbm.at[sdw_v.at[1]], B_v, semB)
            pltpu.async_copy(feat_hbm.at[sdw_v.at[0]], F_v, semF)

    def process(bufs, sems):
        sdw_v, A_v, B_v, EE_v, F_v = bufs
        semA, semB, semF, sem_s, sem_m = sems
        pltpu.make_async_copy(el_hbm.at[sdw_v.at[0]], A_v, semA).wait()
        pltpu.make_async_copy(er_hbm.at[sdw_v.at[1]], B_v, semB).wait()
        pltpu.make_async_copy(feat_hbm.at[sdw_v.at[0]], F_v, semF).wait()

        def fused_group(g, _):
            wg = plsc.bitcast(sdw_v[2, pl.ds(16 * g, 16)], jnp.float32)
            for l in range(16):
                i = 16 * g + l
                e = A_v[i, :] + B_v[i, :]
                e = jnp.maximum(e, jnp.float32(NEG_SLOPE) * e)
                e = e * wg[l]
                ee = jnp.exp(e)
                EE_v[i, :] = ee
                for h in range(D // 16):
                    F_v[i, pl.ds(16 * h, 16)] = F_v[i, pl.ds(16 * h, 16)] * ee[h]
            return 0

        lax.fori_loop(0, K // 16, fused_group, 0)
        pltpu.async_copy(EE_v, s_acc.at[sdw_v.at[1]], sem_s, add=True)
        pltpu.async_copy(F_v, rst_acc.at[sdw_v.at[1]], sem_m, add=True)

    def pair_body(i2, _):
        c0 = 2 * i2

        @pl.when((i2 >= 0) & (c0 < CH))
        def _p0():
            process(bufs0, sems0)

        @pl.when((0 <= c0 + 2) & (c0 + 2 < CH + 2))
        def _prefetch0():
            issue(c0 + 2, bufs0, sems0)

        @pl.when((i2 >= 0) & (c0 + 1 < CH))
        def _p1():
            process(bufs1, sems1)

        @pl.when((0 <= c0 + 3) & (c0 + 3 < CH + 2))
        def _prefetch1():
            issue(c0 + 3, bufs1, sems1)

        return 0

    lax.fori_loop(-1, (CH + 1) // 2 + 1, pair_body, 0)

    plsc.subcore_barrier()
    for r in range(nz):
        sl = pl.ds(base + r * zrows, zrows)
        pltpu.sync_copy(rst_acc.at[sl], rst_out.at[c, sl])
        pltpu.sync_copy(s_acc.at[sl], s_out.at[c, sl])

    @pl.when(s == NS - 1)
    def _copy_tail():
        sl = pl.ds(NS * stripe, rem)
        pltpu.sync_copy(rst_acc.at[sl], rst_out.at[c, sl])
        pltpu.sync_copy(s_acc.at[sl], s_out.at[c, sl])


def kernel(vt, x, edge_index, edge_weight, W_fc, b_fc, attn_l, attn_r, W_out, b_out):
    B, N, Dm = x.shape
    H = attn_l.shape[1]
    DH = attn_l.shape[2]
    OUT = W_out.shape[1]
    E = edge_index.shape[1]
    x2d = x.reshape(N, Dm)

    eye = jnp.eye(H, dtype=jnp.float32)
    al = attn_l.reshape(H, DH)
    ar = attn_r.reshape(H, DH)
    ALRl = (eye[:, None, :] * al[:, :, None]).reshape(H * DH, H)
    ALRl = jnp.concatenate([ALRl, jnp.zeros_like(ALRl)], axis=1)
    ALRr = (eye[:, None, :] * ar[:, :, None]).reshape(H * DH, H)
    ALRr = jnp.concatenate([ALRr, jnp.zeros_like(ALRr)], axis=1)

    bn = 512
    grid1 = (pl.cdiv(N, bn),)
    feat, el16, er16 = pl.pallas_call(
        _t1_body,
        grid=grid1,
        in_specs=[
            pl.BlockSpec((bn, Dm), lambda i: (i, 0)),
            pl.BlockSpec((Dm, Dm), lambda i: (0, 0)),
            pl.BlockSpec((1, Dm), lambda i: (0, 0)),
            pl.BlockSpec((Dm, 16), lambda i: (0, 0)),
            pl.BlockSpec((Dm, 16), lambda i: (0, 0)),
        ],
        out_specs=[
            pl.BlockSpec((bn, Dm), lambda i: (i, 0)),
            pl.BlockSpec((bn, 16), lambda i: (i, 0)),
            pl.BlockSpec((bn, 16), lambda i: (i, 0)),
        ],
        out_shape=[
            jax.ShapeDtypeStruct((N, Dm), jnp.float32),
            jax.ShapeDtypeStruct((N, 16), jnp.float32),
            jax.ShapeDtypeStruct((N, 16), jnp.float32),
        ],
        interpret=_INTERP,
    )(x2d, W_fc.T, b_fc.reshape(1, Dm), ALRl, ALRr)

    TILES = NC * NS
    per_tile = E // TILES
    CH = -(-per_tile // K)
    pt_pad = CH * K
    npad8 = 8
    src2 = jnp.pad(edge_index[0].reshape(TILES, per_tile), ((0, 0), (0, pt_pad - per_tile)),
                   constant_values=N)
    dst2 = jnp.pad(edge_index[1].reshape(TILES, per_tile), ((0, 0), (0, pt_pad - per_tile)),
                   constant_values=N)
    w2 = jnp.pad(edge_weight.reshape(TILES, per_tile), ((0, 0), (0, pt_pad - per_tile)),
                 constant_values=1.0)
    src4 = src2.reshape(NC, NS, CH, K)
    dst4 = dst2.reshape(NC, NS, CH, K)
    w4i = lax.bitcast_convert_type(w2, jnp.int32).reshape(NC, NS, CH, K)
    sdw = jnp.stack([src4, dst4, w4i], axis=3)
    NP = N + npad8
    el16 = jnp.pad(el16, ((0, npad8), (0, 0)), constant_values=-1e30)
    er16 = jnp.pad(er16, ((0, npad8), (0, 0)))
    feat_p = jnp.pad(feat, ((0, npad8), (0, 0)))

    zrows = 104
    mesh = plsc.VectorSubcoreMesh(
        core_axis_name="c", subcore_axis_name="s", num_cores=NC, num_subcores=NS
    )

    def _bufset():
        return (
            pltpu.VMEM((3, K), jnp.int32),
            pltpu.VMEM((K, 16), jnp.float32),
            pltpu.VMEM((K, 16), jnp.float32),
            pltpu.VMEM((K, 16), jnp.float32),
            pltpu.VMEM((K, Dm), jnp.float32),
        )

    def _semset():
        return tuple(pltpu.SemaphoreType.DMA for _ in range(5))

    rst2, s2 = pl.kernel(
        _sc_body,
        out_type=[
            jax.ShapeDtypeStruct((NC, NP, Dm), jnp.float32),
            jax.ShapeDtypeStruct((NC, NP, 16), jnp.float32),
        ],
        mesh=mesh,
        compiler_params=pltpu.CompilerParams(
            use_tc_tiling_on_sc=False, needs_layout_passes=False,

        ),
        scratch_types=[
            _bufset(),
            _bufset(),
            pltpu.VMEM((zrows, Dm), jnp.float32),
            pltpu.VMEM((zrows, 16), jnp.float32),
            pltpu.VMEM_SHARED((NP, Dm), jnp.float32),
            pltpu.VMEM_SHARED((NP, 16), jnp.float32),
            _semset(),
            _semset(),
        ],
    )(sdw, el16, er16, feat_p)

    SEL = (eye[:, None, :] * jnp.ones((H, DH, H), jnp.float32)).reshape(H * DH, H).T
    SEL = jnp.concatenate([SEL, jnp.zeros_like(SEL)], axis=0)
    BD = (eye[:, None, :, None] * W_out[None, :, None, :]).reshape(H * DH, H * OUT)
    bout = jnp.tile(b_out, (H,)).reshape(1, H * OUT)

    grid2 = (pl.cdiv(N, bn),)
    out = pl.pallas_call(
        _t2_body,
        grid=grid2,
        in_specs=[
            pl.BlockSpec((NC, bn, Dm), lambda i: (0, i, 0)),
            pl.BlockSpec((NC, bn, 16), lambda i: (0, i, 0)),
            pl.BlockSpec((16, Dm), lambda i: (0, 0)),
            pl.BlockSpec((Dm, H * OUT), lambda i: (0, 0)),
            pl.BlockSpec((1, H * OUT), lambda i: (0, 0)),
        ],
        out_specs=pl.BlockSpec((bn, H * OUT), lambda i: (i, 0)),
        out_shape=jax.ShapeDtypeStruct((N, H * OUT), jnp.float32),
        interpret=_INTERP,
    )(rst2, s2, SEL, BD, bout)

    return out.reshape(N, H, OUT)[None]


# speedup vs baseline: 1.2187x; 1.2187x over previous
import jax
import jax.numpy as jnp
from jax import lax
from jax.experimental import pallas as pl
from jax.experimental.pallas import tpu as pltpu
from jax.experimental.pallas import tpu_sc as plsc

NEG_SLOPE = 0.1
_INTERP = False

NC = 2
NS = 16
K = 64


def _t1_body(x_ref, wft_ref, b_ref, alrl_ref, alrr_ref, feat_ref, el_ref, er_ref):
    xb = x_ref[...]
    feat = jnp.dot(xb, wft_ref[...], preferred_element_type=jnp.float32) + b_ref[...]
    feat_ref[...] = feat
    el_ref[...] = jnp.dot(feat, alrl_ref[...], preferred_element_type=jnp.float32)
    er_ref[...] = jnp.dot(feat, alrr_ref[...], preferred_element_type=jnp.float32)


def _t2_body(rst_ref, s_ref, sel_ref, bd_ref, bout_ref, out_ref):
    rst = rst_ref[0] + rst_ref[1]
    sv = s_ref[0] + s_ref[1]
    dexp = jnp.dot(sv, sel_ref[...], preferred_element_type=jnp.float32)
    rstn = rst / (dexp + 1e-9)
    out_ref[...] = (
        jnp.dot(rstn, bd_ref[...], preferred_element_type=jnp.float32) + bout_ref[...]
    )


def _sc_body(sdw_hbm, el_hbm, er_hbm, feat_hbm,
             rst_out, s_out,
             bufs0, bufs1, zA, zB,
             rst_acc, s_acc, sems0, sems1):
    N = el_hbm.shape[0]
    D = feat_hbm.shape[1]
    CH = sdw_hbm.shape[2]
    stripe = (N // NS) // 8 * 8
    rem = N - stripe * NS
    zrows = zA.shape[0]
    nz = stripe // zrows
    c = lax.axis_index("c")
    s = lax.axis_index("s")
    base = s * stripe

    def zrow(i, _):
        for j in range(D // 16):
            zA[i, pl.ds(16 * j, 16)] = jnp.zeros((16,), jnp.float32)
        zB[i, :] = jnp.zeros((16,), jnp.float32)
        return 0

    lax.fori_loop(0, zrows, zrow, 0)
    for r in range(nz):
        pltpu.sync_copy(zA, rst_acc.at[pl.ds(base + r * zrows, zrows)])
        pltpu.sync_copy(zB, s_acc.at[pl.ds(base + r * zrows, zrows)])

    @pl.when(s == NS - 1)
    def _zero_tail():
        pltpu.sync_copy(zA.at[pl.ds(0, rem)], rst_acc.at[pl.ds(NS * stripe, rem)])
        pltpu.sync_copy(zB.at[pl.ds(0, rem)], s_acc.at[pl.ds(NS * stripe, rem)])

    plsc.subcore_barrier()

    def issue(ci, bufs, sems):
        sdw_v, A_v, B_v, EE_v, F_v = bufs
        semA, semB, semF, sem_s, sem_m = sems

        @pl.when(ci >= 2)
        def _drain():
            pltpu.make_async_copy(EE_v, s_acc.at[sdw_v.at[1]], sem_s).wait()
            pltpu.make_async_copy(F_v, rst_acc.at[sdw_v.at[1]], sem_m).wait()

        @pl.when(ci < CH)
        def _fetch():
            pltpu.sync_copy(sdw_hbm.at[c, s, ci], sdw_v)
            pltpu.async_copy(el_hbm.at[sdw_v.at[0]], A_v, semA)
            pltpu.async_copy(er_hbm.at[sdw_v.at[1]], B_v, semB)
            pltpu.async_copy(feat_hbm.at[sdw_v.at[0]], F_v, semF)

    def process(bufs, sems):
        sdw_v, A_v, B_v, EE_v, F_v = bufs
        semA, semB, semF, sem_s, sem_m = sems
        pltpu.make_async_copy(el_hbm.at[sdw_v.at[0]], A_v, semA).wait()
        pltpu.make_async_copy(er_hbm.at[sdw_v.at[1]], B_v, semB).wait()

        def ee_group(g, _):
            wg = plsc.bitcast(sdw_v[2, pl.ds(16 * g, 16)], jnp.float32)
            for l in range(16):
                i = 16 * g + l
                e = A_v[i, :] + B_v[i, :]
                e = jnp.maximum(e, jnp.float32(NEG_SLOPE) * e)
                e = e * wg[l]
                EE_v[i, :] = jnp.exp(e)
            return 0

        lax.fori_loop(0, K // 16, ee_group, 0, unroll=2)
        pltpu.async_copy(EE_v, s_acc.at[sdw_v.at[1]], sem_s, add=True)
        pltpu.make_async_copy(feat_hbm.at[sdw_v.at[0]], F_v, semF).wait()

        def msg_edge(i, _):
            eerow = EE_v[i, :]
            for h in range(D // 16):
                F_v[i, pl.ds(16 * h, 16)] = F_v[i, pl.ds(16 * h, 16)] * eerow[h]
            return 0

        lax.fori_loop(0, K, msg_edge, 0)
        pltpu.async_copy(F_v, rst_acc.at[sdw_v.at[1]], sem_m, add=True)

    def pair_body(i2, _):
        c0 = 2 * i2

        @pl.when((i2 >= 0) & (c0 < CH))
        def _p0():
            process(bufs0, sems0)

        @pl.when((0 <= c0 + 2) & (c0 + 2 < CH + 2))
        def _prefetch0():
            issue(c0 + 2, bufs0, sems0)

        @pl.when((i2 >= 0) & (c0 + 1 < CH))
        def _p1():
            process(bufs1, sems1)

        @pl.when((0 <= c0 + 3) & (c0 + 3 < CH + 2))
        def _prefetch1():
            issue(c0 + 3, bufs1, sems1)

        return 0

    lax.fori_loop(-1, (CH + 1) // 2 + 1, pair_body, 0)

    plsc.subcore_barrier()
    for r in range(nz):
        sl = pl.ds(base + r * zrows, zrows)
        pltpu.sync_copy(rst_acc.at[sl], rst_out.at[c, sl])
        pltpu.sync_copy(s_acc.at[sl], s_out.at[c, sl])

    @pl.when(s == NS - 1)
    def _copy_tail():
        sl = pl.ds(NS * stripe, rem)
        pltpu.sync_copy(rst_acc.at[sl], rst_out.at[c, sl])
        pltpu.sync_copy(s_acc.at[sl], s_out.at[c, sl])


def kernel(vt, x, edge_index, edge_weight, W_fc, b_fc, attn_l, attn_r, W_out, b_out):
    B, N, Dm = x.shape
    H = attn_l.shape[1]
    DH = attn_l.shape[2]
    OUT = W_out.shape[1]
    E = edge_index.shape[1]
    x2d = x.reshape(N, Dm)

    eye = jnp.eye(H, dtype=jnp.float32)
    al = attn_l.reshape(H, DH)
    ar = attn_r.reshape(H, DH)
    ALRl = (eye[:, None, :] * al[:, :, None]).reshape(H * DH, H)
    ALRl = jnp.concatenate([ALRl, jnp.zeros_like(ALRl)], axis=1)
    ALRr = (eye[:, None, :] * ar[:, :, None]).reshape(H * DH, H)
    ALRr = jnp.concatenate([ALRr, jnp.zeros_like(ALRr)], axis=1)

    bn = 512
    grid1 = (pl.cdiv(N, bn),)
    feat, el16, er16 = pl.pallas_call(
        _t1_body,
        grid=grid1,
        in_specs=[
            pl.BlockSpec((bn, Dm), lambda i: (i, 0)),
            pl.BlockSpec((Dm, Dm), lambda i: (0, 0)),
            pl.BlockSpec((1, Dm), lambda i: (0, 0)),
            pl.BlockSpec((Dm, 16), lambda i: (0, 0)),
            pl.BlockSpec((Dm, 16), lambda i: (0, 0)),
        ],
        out_specs=[
            pl.BlockSpec((bn, Dm), lambda i: (i, 0)),
            pl.BlockSpec((bn, 16), lambda i: (i, 0)),
            pl.BlockSpec((bn, 16), lambda i: (i, 0)),
        ],
        out_shape=[
            jax.ShapeDtypeStruct((N, Dm), jnp.float32),
            jax.ShapeDtypeStruct((N, 16), jnp.float32),
            jax.ShapeDtypeStruct((N, 16), jnp.float32),
        ],
        interpret=_INTERP,
    )(x2d, W_fc.T, b_fc.reshape(1, Dm), ALRl, ALRr)

    TILES = NC * NS
    per_tile = E // TILES
    CH = -(-per_tile // K)
    pt_pad = CH * K
    npad8 = 8
    src2 = jnp.pad(edge_index[0].reshape(TILES, per_tile), ((0, 0), (0, pt_pad - per_tile)),
                   constant_values=N)
    dst2 = jnp.pad(edge_index[1].reshape(TILES, per_tile), ((0, 0), (0, pt_pad - per_tile)),
                   constant_values=N)
    w2 = jnp.pad(edge_weight.reshape(TILES, per_tile), ((0, 0), (0, pt_pad - per_tile)),
                 constant_values=1.0)
    src4 = src2.reshape(NC, NS, CH, K)
    dst4 = dst2.reshape(NC, NS, CH, K)
    w4i = lax.bitcast_convert_type(w2, jnp.int32).reshape(NC, NS, CH, K)
    sdw = jnp.stack([src4, dst4, w4i], axis=3)
    NP = N + npad8
    el16 = jnp.pad(el16, ((0, npad8), (0, 0)), constant_values=-1e30)
    er16 = jnp.pad(er16, ((0, npad8), (0, 0)))
    feat_p = jnp.pad(feat, ((0, npad8), (0, 0)))

    zrows = 104
    mesh = plsc.VectorSubcoreMesh(
        core_axis_name="c", subcore_axis_name="s", num_cores=NC, num_subcores=NS
    )

    def _bufset():
        return (
            pltpu.VMEM((3, K), jnp.int32),
            pltpu.VMEM((K, 16), jnp.float32),
            pltpu.VMEM((K, 16), jnp.float32),
            pltpu.VMEM((K, 16), jnp.float32),
            pltpu.VMEM((K, Dm), jnp.float32),
        )

    def _semset():
        return tuple(pltpu.SemaphoreType.DMA for _ in range(5))

    rst2, s2 = pl.kernel(
        _sc_body,
        out_type=[
            jax.ShapeDtypeStruct((NC, NP, Dm), jnp.float32),
            jax.ShapeDtypeStruct((NC, NP, 16), jnp.float32),
        ],
        mesh=mesh,
        compiler_params=pltpu.CompilerParams(
            use_tc_tiling_on_sc=False, needs_layout_passes=False,

        ),
        scratch_types=[
            _bufset(),
            _bufset(),
            pltpu.VMEM((zrows, Dm), jnp.float32),
            pltpu.VMEM((zrows, 16), jnp.float32),
            pltpu.VMEM_SHARED((NP, Dm), jnp.float32),
            pltpu.VMEM_SHARED((NP, 16), jnp.float32),
            _semset(),
            _semset(),
        ],
    )(sdw, el16, er16, feat_p)

    SEL = (eye[:, None, :] * jnp.ones((H, DH, H), jnp.float32)).reshape(H * DH, H).T
    SEL = jnp.concatenate([SEL, jnp.zeros_like(SEL)], axis=0)
    BD = (eye[:, None, :, None] * W_out[None, :, None, :]).reshape(H * DH, H * OUT)
    bout = jnp.tile(b_out, (H,)).reshape(1, H * OUT)

    grid2 = (pl.cdiv(N, bn),)
    out = pl.pallas_call(
        _t2_body,
        grid=grid2,
        in_specs=[
            pl.BlockSpec((NC, bn, Dm), lambda i: (0, i, 0)),
            pl.BlockSpec((NC, bn, 16), lambda i: (0, i, 0)),
            pl.BlockSpec((16, Dm), lambda i: (0, 0)),
            pl.BlockSpec((Dm, H * OUT), lambda i: (0, 0)),
            pl.BlockSpec((1, H * OUT), lambda i: (0, 0)),
        ],
        out_specs=pl.BlockSpec((bn, H * OUT), lambda i: (i, 0)),
        out_shape=jax.ShapeDtypeStruct((N, H * OUT), jnp.float32),
        interpret=_INTERP,
    )(rst2, s2, SEL, BD, bout)

    return out.reshape(N, H, OUT)[None]


# speedup vs baseline: 1.2383x; 1.0161x over previous
import jax
import jax.numpy as jnp
from jax import lax
from jax.experimental import pallas as pl
from jax.experimental.pallas import tpu as pltpu
from jax.experimental.pallas import tpu_sc as plsc

NEG_SLOPE = 0.1

NC = 2
NS = 16
K = 64


def _t1_body(x_ref, wft_ref, b_ref, alrl_ref, alrr_ref, feat_ref, el_ref, er_ref):
    xb = x_ref[...]
    feat = jnp.dot(xb, wft_ref[...], preferred_element_type=jnp.float32) + b_ref[...]
    feat_ref[...] = feat
    el_ref[...] = jnp.dot(feat, alrl_ref[...], preferred_element_type=jnp.float32)
    er_ref[...] = jnp.dot(feat, alrr_ref[...], preferred_element_type=jnp.float32)


def _t2_body(rst_ref, s_ref, sel_ref, bd_ref, bout_ref, out_ref):
    rst = rst_ref[0] + rst_ref[1]
    sv = s_ref[0] + s_ref[1]
    dexp = jnp.dot(sv, sel_ref[...], preferred_element_type=jnp.float32)
    rstn = rst / (dexp + 1e-9)
    out_ref[...] = (
        jnp.dot(rstn, bd_ref[...], preferred_element_type=jnp.float32) + bout_ref[...]
    )


def _sc_body(sdw_hbm, el_hbm, er_hbm, feat_hbm,
             rst_out, s_out,
             bufs0, bufs1, zA, zB,
             rst_acc, s_acc, sems0, sems1):
    N = el_hbm.shape[0]
    D = feat_hbm.shape[1]
    CH = sdw_hbm.shape[2]
    stripe = (N // NS) // 8 * 8
    rem = N - stripe * NS
    zrows = zA.shape[0]
    nz = stripe // zrows
    c = lax.axis_index("c")
    s = lax.axis_index("s")
    base = s * stripe

    def zrow(i, _):
        for j in range(D // 16):
            zA[i, pl.ds(16 * j, 16)] = jnp.zeros((16,), jnp.float32)
        zB[i, :] = jnp.zeros((16,), jnp.float32)
        return 0

    lax.fori_loop(0, zrows, zrow, 0)
    for r in range(nz):
        pltpu.sync_copy(zA, rst_acc.at[pl.ds(base + r * zrows, zrows)])
        pltpu.sync_copy(zB, s_acc.at[pl.ds(base + r * zrows, zrows)])

    @pl.when(s == NS - 1)
    def _zero_tail():
        pltpu.sync_copy(zA.at[pl.ds(0, rem)], rst_acc.at[pl.ds(NS * stripe, rem)])
        pltpu.sync_copy(zB.at[pl.ds(0, rem)], s_acc.at[pl.ds(NS * stripe, rem)])

    plsc.subcore_barrier()

    def issue(ci, bufs, sems):
        sdw_v, A_v, B_v, EE_v, F_v = bufs
        semA, semB, semF, sem_s, sem_m = sems

        @pl.when(ci >= 2)
        def _drain():
            pltpu.make_async_copy(EE_v, s_acc.at[sdw_v.at[1]], sem_s).wait()
            pltpu.make_async_copy(F_v, rst_acc.at[sdw_v.at[1]], sem_m).wait()

        @pl.when(ci < CH)
        def _fetch():
            pltpu.sync_copy(sdw_hbm.at[c, s, ci], sdw_v)
            pltpu.async_copy(el_hbm.at[sdw_v.at[0]], A_v, semA)
            pltpu.async_copy(er_hbm.at[sdw_v.at[1]], B_v, semB)
            pltpu.async_copy(feat_hbm.at[sdw_v.at[0]], F_v, semF)

    def process(bufs, sems):
        sdw_v, A_v, B_v, EE_v, F_v = bufs
        semA, semB, semF, sem_s, sem_m = sems
        pltpu.make_async_copy(el_hbm.at[sdw_v.at[0]], A_v, semA).wait()
        pltpu.make_async_copy(er_hbm.at[sdw_v.at[1]], B_v, semB).wait()

        def ee_group(g, _):
            wg = plsc.bitcast(sdw_v[2, pl.ds(16 * g, 16)], jnp.float32)
            for l in range(16):
                i = 16 * g + l
                e = A_v[i, :] + B_v[i, :]
                e = jnp.maximum(e, jnp.float32(NEG_SLOPE) * e)
                e = e * wg[l]
                EE_v[i, :] = jnp.exp(e)
            return 0

        lax.fori_loop(0, K // 16, ee_group, 0)
        pltpu.async_copy(EE_v, s_acc.at[sdw_v.at[1]], sem_s, add=True)
        pltpu.make_async_copy(feat_hbm.at[sdw_v.at[0]], F_v, semF).wait()

        def msg_edge(i, _):
            eerow = EE_v[i, :]
            for h in range(D // 16):
                F_v[i, pl.ds(16 * h, 16)] = F_v[i, pl.ds(16 * h, 16)] * eerow[h]
            return 0

        lax.fori_loop(0, K, msg_edge, 0)
        pltpu.async_copy(F_v, rst_acc.at[sdw_v.at[1]], sem_m, add=True)

    def pair_body(i2, _):
        c0 = 2 * i2

        @pl.when((i2 >= 0) & (c0 < CH))
        def _p0():
            process(bufs0, sems0)

        @pl.when((0 <= c0 + 2) & (c0 + 2 < CH + 2))
        def _prefetch0():
            issue(c0 + 2, bufs0, sems0)

        @pl.when((i2 >= 0) & (c0 + 1 < CH))
        def _p1():
            process(bufs1, sems1)

        @pl.when((0 <= c0 + 3) & (c0 + 3 < CH + 2))
        def _prefetch1():
            issue(c0 + 3, bufs1, sems1)

        return 0

    lax.fori_loop(-1, (CH + 1) // 2 + 1, pair_body, 0)

    plsc.subcore_barrier()
    for r in range(nz):
        sl = pl.ds(base + r * zrows, zrows)
        pltpu.sync_copy(rst_acc.at[sl], rst_out.at[c, sl])
        pltpu.sync_copy(s_acc.at[sl], s_out.at[c, sl])

    @pl.when(s == NS - 1)
    def _copy_tail():
        sl = pl.ds(NS * stripe, rem)
        pltpu.sync_copy(rst_acc.at[sl], rst_out.at[c, sl])
        pltpu.sync_copy(s_acc.at[sl], s_out.at[c, sl])


def kernel(vt, x, edge_index, edge_weight, W_fc, b_fc, attn_l, attn_r, W_out, b_out):
    B, N, Dm = x.shape
    H = attn_l.shape[1]
    DH = attn_l.shape[2]
    OUT = W_out.shape[1]
    E = edge_index.shape[1]
    x2d = x.reshape(N, Dm)

    eye = jnp.eye(H, dtype=jnp.float32)
    al = attn_l.reshape(H, DH)
    ar = attn_r.reshape(H, DH)
    ALRl = (eye[:, None, :] * al[:, :, None]).reshape(H * DH, H)
    ALRl = jnp.concatenate([ALRl, jnp.zeros_like(ALRl)], axis=1)
    ALRr = (eye[:, None, :] * ar[:, :, None]).reshape(H * DH, H)
    ALRr = jnp.concatenate([ALRr, jnp.zeros_like(ALRr)], axis=1)

    bn = 512
    grid1 = (pl.cdiv(N, bn),)
    feat, el16, er16 = pl.pallas_call(
        _t1_body,
        grid=grid1,
        in_specs=[
            pl.BlockSpec((bn, Dm), lambda i: (i, 0)),
            pl.BlockSpec((Dm, Dm), lambda i: (0, 0)),
            pl.BlockSpec((1, Dm), lambda i: (0, 0)),
            pl.BlockSpec((Dm, 16), lambda i: (0, 0)),
            pl.BlockSpec((Dm, 16), lambda i: (0, 0)),
        ],
        out_specs=[
            pl.BlockSpec((bn, Dm), lambda i: (i, 0)),
            pl.BlockSpec((bn, 16), lambda i: (i, 0)),
            pl.BlockSpec((bn, 16), lambda i: (i, 0)),
        ],
        out_shape=[
            jax.ShapeDtypeStruct((N, Dm), jnp.float32),
            jax.ShapeDtypeStruct((N, 16), jnp.float32),
            jax.ShapeDtypeStruct((N, 16), jnp.float32),
        ],
    )(x2d, W_fc.T, b_fc.reshape(1, Dm), ALRl, ALRr)

    TILES = NC * NS
    per_tile = E // TILES
    CH = -(-per_tile // K)
    pt_pad = CH * K
    npad8 = 8
    src2 = jnp.pad(edge_index[0].reshape(TILES, per_tile), ((0, 0), (0, pt_pad - per_tile)),
                   constant_values=N)
    dst2 = jnp.pad(edge_index[1].reshape(TILES, per_tile), ((0, 0), (0, pt_pad - per_tile)),
                   constant_values=N)
    w2 = jnp.pad(edge_weight.reshape(TILES, per_tile), ((0, 0), (0, pt_pad - per_tile)),
                 constant_values=1.0)
    src4 = src2.reshape(NC, NS, CH, K)
    dst4 = dst2.reshape(NC, NS, CH, K)
    w4i = lax.bitcast_convert_type(w2, jnp.int32).reshape(NC, NS, CH, K)
    sdw = jnp.stack([src4, dst4, w4i], axis=3)
    NP = N + npad8
    el16 = jnp.pad(el16, ((0, npad8), (0, 0)), constant_values=-1e30)
    er16 = jnp.pad(er16, ((0, npad8), (0, 0)))
    feat_p = jnp.pad(feat, ((0, npad8), (0, 0)))

    zrows = 104
    mesh = plsc.VectorSubcoreMesh(
        core_axis_name="c", subcore_axis_name="s", num_cores=NC, num_subcores=NS
    )

    def _bufset():
        return (
            pltpu.VMEM((3, K), jnp.int32),
            pltpu.VMEM((K, 16), jnp.float32),
            pltpu.VMEM((K, 16), jnp.float32),
            pltpu.VMEM((K, 16), jnp.float32),
            pltpu.VMEM((K, Dm), jnp.float32),
        )

    def _semset():
        return tuple(pltpu.SemaphoreType.DMA for _ in range(5))

    rst2, s2 = pl.kernel(
        _sc_body,
        out_type=[
            jax.ShapeDtypeStruct((NC, NP, Dm), jnp.float32),
            jax.ShapeDtypeStruct((NC, NP, 16), jnp.float32),
        ],
        mesh=mesh,
        compiler_params=pltpu.CompilerParams(
            use_tc_tiling_on_sc=False, needs_layout_passes=False,

        ),
        scratch_types=[
            _bufset(),
            _bufset(),
            pltpu.VMEM((zrows, Dm), jnp.float32),
            pltpu.VMEM((zrows, 16), jnp.float32),
            pltpu.VMEM_SHARED((NP, Dm), jnp.float32),
            pltpu.VMEM_SHARED((NP, 16), jnp.float32),
            _semset(),
            _semset(),
        ],
    )(sdw, el16, er16, feat_p)

    SEL = (eye[:, None, :] * jnp.ones((H, DH, H), jnp.float32)).reshape(H * DH, H).T
    SEL = jnp.concatenate([SEL, jnp.zeros_like(SEL)], axis=0)
    BD = (eye[:, None, :, None] * W_out[None, :, None, :]).reshape(H * DH, H * OUT)
    bout = jnp.tile(b_out, (H,)).reshape(1, H * OUT)

    grid2 = (pl.cdiv(N, bn),)
    out = pl.pallas_call(
        _t2_body,
        grid=grid2,
        in_specs=[
            pl.BlockSpec((NC, bn, Dm), lambda i: (0, i, 0)),
            pl.BlockSpec((NC, bn, 16), lambda i: (0, i, 0)),
            pl.BlockSpec((16, Dm), lambda i: (0, 0)),
            pl.BlockSpec((Dm, H * OUT), lambda i: (0, 0)),
            pl.BlockSpec((1, H * OUT), lambda i: (0, 0)),
        ],
        out_specs=pl.BlockSpec((bn, H * OUT), lambda i: (i, 0)),
        out_shape=jax.ShapeDtypeStruct((N, H * OUT), jnp.float32),
    )(rst2, s2, SEL, BD, bout)

    return out.reshape(N, H, OUT)[None]
